# baseline (speedup 1.0000x reference)
import functools

import jax
import jax.numpy as jnp
from jax import lax
from jax.experimental import pallas as pl
from jax.experimental.pallas import tpu as pltpu
from jax.experimental.pallas import tpu_sc as plsc

VOCAB = 64
EMBED_DIM = 64
BATCH = 4096
SEQ = 50

_NW = 32
_B_W = BATCH // _NW
_NG = _B_W // 16
_SC = 2
_N_CH = SEQ // _SC


def _sc_lookup(table_rep, ids_t):
    mesh = plsc.VectorSubcoreMesh(core_axis_name="c", subcore_axis_name="s")

    @functools.partial(
        pl.kernel,
        out_type=jax.ShapeDtypeStruct((SEQ, EMBED_DIM, BATCH), jnp.float32),
        mesh=mesh,
        scratch_types=[
            pltpu.VMEM((SEQ, _B_W), jnp.int32),
            pltpu.VMEM((VOCAB * EMBED_DIM * 16,), jnp.float32),
            pltpu.VMEM((_SC, EMBED_DIM, _B_W), jnp.float32),
            pltpu.VMEM((_SC, EMBED_DIM, _B_W), jnp.float32),
            pltpu.SemaphoreType.DMA,
            pltpu.SemaphoreType.DMA,
            pltpu.SemaphoreType.DMA,
        ],
        compiler_params=pltpu.CompilerParams(use_tc_tiling_on_sc=False,
                                             needs_layout_passes=False),
    )
    def k(tab_hbm, ids_hbm, out_hbm, ids_v, tab_v, buf0, buf1, s0, s1, si_sem):
        wid = lax.axis_index("s") * 2 + lax.axis_index("c")
        b0 = wid * _B_W
        pltpu.sync_copy(tab_hbm, tab_v)
        pltpu.async_copy(ids_hbm.at[:, pl.ds(b0, _B_W)], ids_v, si_sem).wait()

        lane = lax.iota(jnp.int32, 16)

        def compute(ch, buf):
            @plsc.parallel_loop(0, _NG, 1, unroll=1)
            def _(bg):
                for si in range(_SC):
                    srow = jnp.full((16,), ch * _SC + si, jnp.int32)
                    vidx = plsc.load_gather(ids_v, [srow, bg * 16 + lane])
                    abase = lax.shift_left(vidx, 4) + lane
                    for d in range(EMBED_DIM):
                        vals = plsc.load_gather(
                            tab_v, [abase + (d * VOCAB * 16)])
                        buf[si, d, pl.ds(bg * 16, 16)] = vals

        def start_store(ch, buf, sem):
            pltpu.async_copy(
                buf, out_hbm.at[pl.ds(ch * _SC, _SC), :, pl.ds(b0, _B_W)],
                sem)

        def wait_store(buf, sem):
            pltpu.make_async_copy(
                buf, out_hbm.at[pl.ds(0, _SC), :, pl.ds(b0, _B_W)],
                sem).wait()

        compute(0, buf0)
        start_store(0, buf0, s0)

        @pl.loop(1, _N_CH, step=2)
        def _(ch):
            @pl.when(ch > 1)
            def _():
                wait_store(buf1, s1)
            compute(ch, buf1)
            start_store(ch, buf1, s1)
            wait_store(buf0, s0)
            compute(ch + 1, buf0)
            start_store(ch + 1, buf0, s0)

        wait_store(buf0, s0)
        wait_store(buf1, s1)

    return k(table_rep, ids_t)


def kernel(input_ids, token_embedding_table):
    ids_t = input_ids.T
    rep = jnp.broadcast_to(token_embedding_table.T[:, :, None],
                           (EMBED_DIM, VOCAB, 16))
    out = _sc_lookup(rep.reshape(EMBED_DIM * VOCAB * 16), ids_t)
    return jnp.transpose(out, (2, 0, 1))


# speedup vs baseline: 1.1976x; 1.1976x over previous
import functools

import jax
import jax.numpy as jnp
from jax import lax
from jax.experimental import pallas as pl
from jax.experimental.pallas import tpu as pltpu
from jax.experimental.pallas import tpu_sc as plsc

VOCAB = 64
EMBED_DIM = 64
BATCH = 4096
SEQ = 50

_B = BATCH * SEQ
_NW = 32
_B_PER_W = _B // _NW
_CHUNK = 640
_N_CHUNKS = _B_PER_W // _CHUNK
_G = _CHUNK // 16


def _sc_gather(table_flat, ids_flat):
    mesh = plsc.VectorSubcoreMesh(core_axis_name="c", subcore_axis_name="s")

    @functools.partial(
        pl.kernel,
        out_type=jax.ShapeDtypeStruct((_B * EMBED_DIM,), jnp.float32),
        mesh=mesh,
        scratch_types=[
            pltpu.VMEM((_B_PER_W,), jnp.int32),
            pltpu.VMEM((VOCAB * EMBED_DIM,), jnp.float32),
            pltpu.VMEM((_CHUNK * EMBED_DIM,), jnp.float32),
            pltpu.VMEM((_CHUNK * EMBED_DIM,), jnp.float32),
            pltpu.SemaphoreType.DMA,
            pltpu.SemaphoreType.DMA,
        ],
        compiler_params=pltpu.CompilerParams(use_tc_tiling_on_sc=True,
                                             needs_layout_passes=False),
    )
    def k(table_hbm, idx_hbm, out_hbm, idx_v, table_v, rows0, rows1, s0, s1):
        wid = lax.axis_index("s") * 2 + lax.axis_index("c")
        base = wid * _B_PER_W
        rows = [rows0, rows1]
        ssem = [s0, s1]
        pltpu.sync_copy(table_hbm, table_v)
        pltpu.sync_copy(idx_hbm.at[pl.ds(base, _B_PER_W)], idx_v)

        lane = lax.iota(jnp.int32, 16)
        dst0 = lane * EMBED_DIM

        def splat_lane(vec, i):
            return lax.gather(
                vec, jnp.full((16, 1), i, jnp.int32),
                lax.GatherDimensionNumbers(
                    offset_dims=(), collapsed_slice_dims=(0,),
                    start_index_map=(0,)),
                slice_sizes=(1,),
                mode=lax.GatherScatterMode.PROMISE_IN_BOUNDS)

        lane_k = [lane + 16 * kk for kk in range(EMBED_DIM // 16)]

        def compute(ch, rows_b):
            @plsc.parallel_loop(0, _G, 1, unroll=1)
            def _(gl):
                vidx = idx_v[pl.ds(ch * _CHUNK + gl * 16, 16)]
                svec = lax.shift_left(vidx, 6)
                obase = gl * (16 * EMBED_DIM)
                for i in range(16):
                    rbase = splat_lane(svec, i)
                    for kk in range(EMBED_DIM // 16):
                        vals = plsc.load_gather(table_v, [rbase + lane_k[kk]])
                        rows_b[pl.ds(obase + i * EMBED_DIM + 16 * kk, 16)] = vals

        def start_store(ch, rows_b, sem):
            pltpu.async_copy(
                rows_b,
                out_hbm.at[pl.ds((base + ch * _CHUNK) * EMBED_DIM,
                                 _CHUNK * EMBED_DIM)],
                sem)

        def wait_store(rows_b, sem):
            pltpu.make_async_copy(
                rows_b,
                out_hbm.at[pl.ds(base * EMBED_DIM, _CHUNK * EMBED_DIM)],
                sem).wait()

        compute(0, rows0)
        start_store(0, rows0, s0)
        compute(1, rows1)
        start_store(1, rows1, s1)

        @pl.loop(2, _N_CHUNKS, step=2)
        def _(ch):
            wait_store(rows0, s0)
            compute(ch, rows0)
            start_store(ch, rows0, s0)
            wait_store(rows1, s1)
            compute(ch + 1, rows1)
            start_store(ch + 1, rows1, s1)

        wait_store(rows0, s0)
        wait_store(rows1, s1)

    return k(table_flat, ids_flat)


def kernel(input_ids, token_embedding_table):
    ids_flat = jnp.where(input_ids < 0, 0, input_ids).reshape(_B)
    table_flat = token_embedding_table.reshape(VOCAB * EMBED_DIM)
    out = _sc_gather(table_flat, ids_flat)
    return out.reshape(BATCH, SEQ, EMBED_DIM)
